# Initial kernel scaffold; baseline (speedup 1.0000x reference)
import jax
import jax.numpy as jnp
from jax.experimental import pallas as pl


def kernel(cpoint_loc, alpha):
    raise NotImplementedError("write your pallas kernel here")


# speedup vs baseline: 119.4316x; 119.4316x over previous
import jax
import jax.numpy as jnp
from jax.experimental import pallas as pl
from jax.experimental.pallas import tpu as pltpu

I_SIZE = 512
BATCH = 8
NPOINT = 160
RM = 37
NWIN = 2 * RM
ROWS = 88
COLS = 256


def _radius_kernel(cp_ref, r_ref):
    cp = cp_ref[...]
    x = cp[:, :, 0]
    y = cp[:, :, 1]
    dx = x[:, :, None] - x[:, None, :]
    dy = y[:, :, None] - y[:, None, :]
    sq = dx * dx + dy * dy
    i = jax.lax.broadcasted_iota(jnp.int32, (BATCH, NPOINT, NPOINT), 1)
    j = jax.lax.broadcasted_iota(jnp.int32, (BATCH, NPOINT, NPOINT), 2)
    sq = sq + jnp.where(i == j, jnp.float32(1e12), jnp.float32(0.0))
    d = jnp.sqrt(sq)
    dmin = jnp.min(d, axis=2)
    r_ref[0, :] = jnp.max(dmin, axis=1) * jnp.float32(2.0)


def _splat_kernel(ib_ref, fs_ref, r_ref, rmax_ref, wcol_ref, wrow_ref,
                  out_ref):
    b = pl.program_id(0)
    inv_r = jnp.float32(1.0) / r_ref[0, b]
    r_max = rmax_ref[0, 0]
    out_ref[...] = jnp.zeros_like(out_ref)

    row_iota = jax.lax.broadcasted_iota(jnp.int32, (ROWS, COLS), 0).astype(
        jnp.float32)
    col_iota = jax.lax.broadcasted_iota(jnp.int32, (ROWS, COLS), 1).astype(
        jnp.float32)
    rows1 = jax.lax.broadcasted_iota(jnp.int32, (ROWS, 1), 0).astype(
        jnp.float32)
    cols1 = jax.lax.broadcasted_iota(jnp.int32, (1, COLS), 1).astype(
        jnp.float32)

    wcol = wcol_ref[:, 0:1]
    wrow = wrow_ref[0:1, :]
    iw_col = (wcol >= -r_max) & (wcol <= r_max - 1.0)
    iw_row = (wrow >= -r_max) & (wrow <= r_max - 1.0)

    def body(n, _):
        yb = pl.multiple_of(ib_ref[0, b, n], 8)
        xb = pl.multiple_of(ib_ref[1, b, n], 128)
        cx = fs_ref[0, b, n]
        cy = fs_ref[1, b, n]
        t0 = fs_ref[2, b, n]
        t1 = fs_ref[3, b, n]
        ax = fs_ref[4, b, n]
        ay = fs_ref[5, b, n]

        sx = wcol + t0
        sy = wrow + t1
        xs1 = jnp.float32(xb) + cols1
        ys1 = jnp.float32(yb) + rows1
        mx = jnp.sum(
            jnp.where(iw_col & (sx >= xs1) & (sx < xs1 + 1.0),
                      jnp.float32(1.0), jnp.float32(0.0)),
            axis=0, keepdims=True)
        my = jnp.sum(
            jnp.where(iw_row & (sy >= ys1) & (sy < ys1 + 1.0),
                      jnp.float32(1.0), jnp.float32(0.0)),
            axis=1, keepdims=True)

        xs = jnp.float32(xb) + col_iota
        ys = jnp.float32(yb) + row_iota
        dxp = xs - cx
        dyp = ys - cy
        dist = jnp.sqrt(dxp * dxp + dyp * dyp + jnp.float32(1e-12)) * inv_r
        u = jnp.float32(1.0) - dist
        u2 = u * u
        psi = jnp.where(dist < 1.0,
                        u2 * u2 * (jnp.float32(4.0) * dist + 1.0),
                        jnp.float32(0.0))
        w = psi * (mx * my)
        out_ref[0, 0, pl.ds(yb, ROWS), pl.ds(xb, COLS)] += w * ax
        out_ref[0, 1, pl.ds(yb, ROWS), pl.ds(xb, COLS)] += w * ay
        return 0

    jax.lax.fori_loop(0, NPOINT, body, 0)


@jax.jit
def kernel(cpoint_loc, alpha):
    r = pl.pallas_call(
        _radius_kernel,
        out_shape=jax.ShapeDtypeStruct((1, BATCH), jnp.float32),
    )(cpoint_loc)[0]
    r_max = jnp.minimum(jnp.ceil(jnp.max(r)), jnp.float32(RM))
    r_max_i = r_max.astype(jnp.int32)

    rm_f = jnp.float32(RM)
    win = jnp.linspace(-rm_f, rm_f - 1.0, NWIN).astype(jnp.float32)
    wcol = jnp.full((80, 128), 1e9, jnp.float32).at[:NWIN, 0].set(win)
    wrow = jnp.full((8, 128), 1e9, jnp.float32).at[0, :NWIN].set(win)

    c0 = cpoint_loc[..., 0]
    c1 = cpoint_loc[..., 1]
    t0 = jnp.clip(c0, r_max, jnp.float32(I_SIZE) - r_max)
    t1 = jnp.clip(c1, r_max, jnp.float32(I_SIZE) - r_max)
    fx_i = jnp.floor(t0).astype(jnp.int32)
    fy_i = jnp.floor(t1).astype(jnp.int32)
    x_base = jnp.minimum(((fx_i - r_max_i) // 128) * 128,
                         jnp.int32(I_SIZE - COLS))
    y_base = jnp.minimum(((fy_i - r_max_i) // 8) * 8,
                         jnp.int32(I_SIZE - ROWS))
    ibases = jnp.stack([y_base, x_base], axis=0)
    fscal = jnp.stack([c0, c1, t0, t1,
                       alpha[..., 0], alpha[..., 1]], axis=0)

    out = pl.pallas_call(
        _splat_kernel,
        grid=(BATCH,),
        in_specs=[
            pl.BlockSpec(memory_space=pltpu.SMEM),
            pl.BlockSpec(memory_space=pltpu.SMEM),
            pl.BlockSpec(memory_space=pltpu.SMEM),
            pl.BlockSpec(memory_space=pltpu.SMEM),
            pl.BlockSpec((80, 128), lambda b: (0, 0)),
            pl.BlockSpec((8, 128), lambda b: (0, 0)),
        ],
        out_specs=pl.BlockSpec((1, 2, I_SIZE, I_SIZE),
                               lambda b: (b, 0, 0, 0)),
        out_shape=jax.ShapeDtypeStruct((BATCH, 2, I_SIZE, I_SIZE),
                                       jnp.float32),
    )(ibases, fscal, r.reshape(1, BATCH), r_max.reshape(1, 1), wcol, wrow)
    return out
